# Initial kernel scaffold; baseline (speedup 1.0000x reference)
import jax
import jax.numpy as jnp
from jax.experimental import pallas as pl


def kernel(x, table):
    raise NotImplementedError("write your pallas kernel here")


# speedup vs baseline: 5.8146x; 5.8146x over previous
import functools

import jax
import jax.numpy as jnp
from jax import lax
from jax.experimental import pallas as pl
from jax.experimental.pallas import tpu as pltpu
from jax.experimental.pallas import tpu_sc as plsc

NUM_CORES = 2
NUM_SUBCORES = 16
LANES = 16
NW = NUM_CORES * NUM_SUBCORES

N_POS = 1024 * 16 * 16
D = 32
ROWS = 300
PER_TEC = N_POS // NW
CHUNK = 1024
N_CHUNKS = PER_TEC // CHUNK
GROUPS = CHUNK // LANES

_mesh = plsc.VectorSubcoreMesh(
    core_axis_name="c", subcore_axis_name="s",
    num_cores=NUM_CORES, num_subcores=NUM_SUBCORES,
)


@functools.partial(
    pl.kernel,
    out_type=jax.ShapeDtypeStruct((N_POS * D,), jnp.float32),
    mesh=_mesh,
    scratch_types=[
        pltpu.VMEM((ROWS * D,), jnp.float32),
        pltpu.VMEM((3, CHUNK), jnp.int32),
        pltpu.VMEM((CHUNK * D,), jnp.float32),
    ],
    compiler_params=pltpu.CompilerParams(needs_layout_passes=False),
)
def _sc_encode(x_hbm, table_hbm, out_hbm, table_v, x_v, out_v):
    wid = lax.axis_index("s") * NUM_CORES + lax.axis_index("c")
    pltpu.sync_copy(table_hbm, table_v)
    lane = lax.iota(jnp.int32, LANES)

    def chunk_body(chunk, carry):
        base = wid * PER_TEC + chunk * CHUNK
        pltpu.sync_copy(x_hbm.at[:, pl.ds(base, CHUNK)], x_v)

        def group_body(g, carry):
            goff = g * LANES
            b0 = x_v[0, pl.ds(goff, LANES)] * D
            b1 = x_v[1, pl.ds(goff, LANES)] * D + 100 * D
            b2 = x_v[2, pl.ds(goff, LANES)] * D + 200 * D
            ob = (goff + lane) * D
            for d in range(D):
                s = (plsc.load_gather(table_v, [b0 + d])
                     + plsc.load_gather(table_v, [b1 + d])
                     + plsc.load_gather(table_v, [b2 + d]))
                plsc.store_scatter(out_v, [ob + d], s)
            return carry

        lax.fori_loop(0, GROUPS, group_body, 0)
        pltpu.sync_copy(out_v, out_hbm.at[pl.ds(base * D, CHUNK * D)])
        return carry

    lax.fori_loop(0, N_CHUNKS, chunk_body, 0)


def kernel(x, table):
    xt = x.reshape(-1, 3).T
    return _sc_encode(xt, table.reshape(-1))


# speedup vs baseline: 14.5403x; 2.5006x over previous
import functools

import numpy as _np
import jax
import jax.numpy as jnp
from jax import lax
from jax.experimental import pallas as pl
from jax.experimental.pallas import tpu as pltpu
from jax.experimental.pallas import tpu_sc as plsc

NUM_CORES = 2
NUM_SUBCORES = 16
LANES = 16
NW = NUM_CORES * NUM_SUBCORES

N_POS = 1024 * 16 * 16
D = 32
ROWS = 300
PER_TEC = N_POS // NW
CHUNK = 1024
N_CHUNKS = PER_TEC // CHUNK
GROUPS = CHUNK // LANES

_mesh = plsc.VectorSubcoreMesh(
    core_axis_name="c", subcore_axis_name="s",
    num_cores=NUM_CORES, num_subcores=NUM_SUBCORES,
)


@functools.partial(
    pl.kernel,
    out_type=jax.ShapeDtypeStruct((N_POS * D,), jnp.float32),
    mesh=_mesh,
    scratch_types=[
        pltpu.VMEM((ROWS * D,), jnp.float32),
        pltpu.VMEM((3, CHUNK), jnp.int32),
        pltpu.VMEM((CHUNK * D,), jnp.float32),
        pltpu.VMEM((D * LANES,), jnp.int32),
    ],
    compiler_params=pltpu.CompilerParams(needs_layout_passes=False),
)
def _sc_encode(x_hbm, table_hbm, diag_hbm, out_hbm, table_v, x_v, out_v, diag_v):
    wid = lax.axis_index("s") * NUM_CORES + lax.axis_index("c")
    pltpu.sync_copy(table_hbm, table_v)
    pltpu.sync_copy(diag_hbm, diag_v)
    lane = lax.iota(jnp.int32, LANES)

    def chunk_body(chunk, carry):
        base = wid * PER_TEC + chunk * CHUNK
        pltpu.sync_copy(x_hbm.at[:, pl.ds(base, CHUNK)], x_v)

        def group_body(g, carry):
            goff = g * LANES
            b0 = x_v[0, pl.ds(goff, LANES)] * D
            b1 = x_v[1, pl.ds(goff, LANES)] * D + 100 * D
            b2 = x_v[2, pl.ds(goff, LANES)] * D + 200 * D
            ob = (goff + lane) * D
            for k in range(D):
                dv = diag_v[pl.ds(k * LANES, LANES)]
                s = (plsc.load_gather(table_v, [b0 + dv])
                     + plsc.load_gather(table_v, [b1 + dv])
                     + plsc.load_gather(table_v, [b2 + dv]))
                plsc.store_scatter(out_v, [ob + dv], s)
            return carry

        lax.fori_loop(0, GROUPS, group_body, 0)
        pltpu.sync_copy(out_v, out_hbm.at[pl.ds(base * D, CHUNK * D)])
        return carry

    lax.fori_loop(0, N_CHUNKS, chunk_body, 0)


_DIAG = _np.asarray(
    [(l + k) % D for k in range(D) for l in range(LANES)], dtype=_np.int32)


def kernel(x, table):
    xt = x.reshape(-1, 3).T
    return _sc_encode(xt, table.reshape(-1), jnp.asarray(_DIAG))


# speedup vs baseline: 29.8510x; 2.0530x over previous
import functools

import numpy as _np
import jax
import jax.numpy as jnp
from jax import lax
from jax.experimental import pallas as pl
from jax.experimental.pallas import tpu as pltpu
from jax.experimental.pallas import tpu_sc as plsc

NUM_CORES = 2
NUM_SUBCORES = 16
LANES = 16
NW = NUM_CORES * NUM_SUBCORES

N_POS = 1024 * 16 * 16
D = 32
ROWS = 300
PER_TEC = N_POS // NW
CHUNK = 1024
N_CHUNKS = PER_TEC // CHUNK
GROUPS = CHUNK // LANES

_mesh = plsc.VectorSubcoreMesh(
    core_axis_name="c", subcore_axis_name="s",
    num_cores=NUM_CORES, num_subcores=NUM_SUBCORES,
)


@functools.partial(
    pl.kernel,
    out_type=jax.ShapeDtypeStruct((N_POS * D,), jnp.float32),
    mesh=_mesh,
    scratch_types=[
        pltpu.VMEM((ROWS * D,), jnp.float32),
        pltpu.VMEM((3, CHUNK), jnp.int32),
        pltpu.VMEM((CHUNK * D,), jnp.float32),
        pltpu.VMEM((D * LANES,), jnp.int32),
    ],
    compiler_params=pltpu.CompilerParams(needs_layout_passes=False),
)
def _sc_encode(x_hbm, table_hbm, diag_hbm, out_hbm, table_v, x_v, out_v, diag_v):
    wid = lax.axis_index("s") * NUM_CORES + lax.axis_index("c")
    pltpu.sync_copy(table_hbm, table_v)
    pltpu.sync_copy(diag_hbm, diag_v)
    lane = lax.iota(jnp.int32, LANES)

    def chunk_body(chunk, carry):
        base = wid * PER_TEC + chunk * CHUNK
        pltpu.sync_copy(x_hbm.at[:, pl.ds(base, CHUNK)], x_v)

        @plsc.parallel_loop(0, GROUPS, unroll=2)
        def group_body(g):
            goff = g * LANES
            b0 = x_v[0, pl.ds(goff, LANES)] * D
            b1 = x_v[1, pl.ds(goff, LANES)] * D + 100 * D
            b2 = x_v[2, pl.ds(goff, LANES)] * D + 200 * D
            ob = (goff + lane) * D
            for k in range(D):
                dv = diag_v[pl.ds(k * LANES, LANES)]
                s = (plsc.load_gather(table_v, [b0 + dv])
                     + plsc.load_gather(table_v, [b1 + dv])
                     + plsc.load_gather(table_v, [b2 + dv]))
                plsc.store_scatter(out_v, [ob + dv], s)

        pltpu.sync_copy(out_v, out_hbm.at[pl.ds(base * D, CHUNK * D)])
        return carry

    lax.fori_loop(0, N_CHUNKS, chunk_body, 0)


_DIAG = _np.asarray(
    [(l + k) % D for k in range(D) for l in range(LANES)], dtype=_np.int32)


def kernel(x, table):
    xt = x.reshape(-1, 3).T
    return _sc_encode(xt, table.reshape(-1), jnp.asarray(_DIAG))


# speedup vs baseline: 32.8746x; 1.1013x over previous
import functools

import numpy as _np
import jax
import jax.numpy as jnp
from jax import lax
from jax.experimental import pallas as pl
from jax.experimental.pallas import tpu as pltpu
from jax.experimental.pallas import tpu_sc as plsc

NUM_CORES = 2
NUM_SUBCORES = 16
LANES = 16
NW = NUM_CORES * NUM_SUBCORES

N_POS = 1024 * 16 * 16
D = 32
ROWS = 300
PER_TEC = N_POS // NW
CHUNK = 1024
N_CHUNKS = PER_TEC // CHUNK
GROUPS = CHUNK // LANES

_mesh = plsc.VectorSubcoreMesh(
    core_axis_name="c", subcore_axis_name="s",
    num_cores=NUM_CORES, num_subcores=NUM_SUBCORES,
)


@functools.partial(
    pl.kernel,
    out_type=jax.ShapeDtypeStruct((N_POS * D,), jnp.float32),
    mesh=_mesh,
    scratch_types=[
        pltpu.VMEM((ROWS * D,), jnp.float32),
        pltpu.VMEM((D * LANES,), jnp.int32),
        pltpu.VMEM((3, CHUNK), jnp.int32),
        pltpu.VMEM((3, CHUNK), jnp.int32),
        pltpu.VMEM((CHUNK * D,), jnp.float32),
        pltpu.VMEM((CHUNK * D,), jnp.float32),
        pltpu.SemaphoreType.DMA,
        pltpu.SemaphoreType.DMA,
        pltpu.SemaphoreType.DMA,
        pltpu.SemaphoreType.DMA,
    ],
    compiler_params=pltpu.CompilerParams(needs_layout_passes=False),
)
def _sc_encode(x_hbm, table_hbm, diag_hbm, out_hbm, table_v, diag_v,
               x_v0, x_v1, out_v0, out_v1, sx0, sx1, so0, so1):
    wid = lax.axis_index("s") * NUM_CORES + lax.axis_index("c")
    pltpu.sync_copy(table_hbm, table_v)
    pltpu.sync_copy(diag_hbm, diag_v)
    lane = lax.iota(jnp.int32, LANES)
    x_bufs = (x_v0, x_v1)
    out_bufs = (out_v0, out_v1)
    x_sems = (sx0, sx1)
    out_sems = (so0, so1)

    def start_x(chunk, b):
        base = wid * PER_TEC + chunk * CHUNK
        pltpu.async_copy(x_hbm.at[:, pl.ds(base, CHUNK)], x_bufs[b], x_sems[b])

    def wait_x(b):
        pltpu.make_async_copy(
            x_hbm.at[:, pl.ds(0, CHUNK)], x_bufs[b], x_sems[b]).wait()

    def start_out(chunk, b):
        base = wid * PER_TEC + chunk * CHUNK
        pltpu.async_copy(
            out_bufs[b], out_hbm.at[pl.ds(base * D, CHUNK * D)], out_sems[b])

    def wait_out(b):
        pltpu.make_async_copy(
            out_bufs[b], out_hbm.at[pl.ds(0, CHUNK * D)], out_sems[b]).wait()

    def compute(b):
        x_v, out_v = x_bufs[b], out_bufs[b]

        @plsc.parallel_loop(0, GROUPS, unroll=2)
        def group_body(g):
            goff = g * LANES
            b0 = x_v[0, pl.ds(goff, LANES)] * D
            b1 = x_v[1, pl.ds(goff, LANES)] * D + 100 * D
            b2 = x_v[2, pl.ds(goff, LANES)] * D + 200 * D
            ob = (goff + lane) * D
            for k in range(D):
                dv = diag_v[pl.ds(k * LANES, LANES)]
                s = (plsc.load_gather(table_v, [b0 + dv])
                     + plsc.load_gather(table_v, [b1 + dv])
                     + plsc.load_gather(table_v, [b2 + dv]))
                plsc.store_scatter(out_v, [ob + dv], s)

    start_x(0, 0)
    start_x(1, 1)

    def pair_body(p, carry):
        c0 = p * 2

        def half(b):
            wait_x(b)

            @pl.when(p > 0)
            def _():
                wait_out(b)

            compute(b)
            start_out(c0 + b, b)

            @pl.when(p < (N_CHUNKS // 2) - 1)
            def _():
                start_x(c0 + 2 + b, b)

        half(0)
        half(1)
        return carry

    lax.fori_loop(0, N_CHUNKS // 2, pair_body, 0)
    wait_out(0)
    wait_out(1)


_DIAG = _np.asarray(
    [(l + k) % D for k in range(D) for l in range(LANES)], dtype=_np.int32)


def kernel(x, table):
    xt = x.reshape(-1, 3).T
    return _sc_encode(xt, table.reshape(-1), jnp.asarray(_DIAG))
